# baseline (speedup 1.0000x reference)
import jax
import jax.numpy as jnp
from jax.experimental import pallas as pl
from jax.experimental.pallas import tpu as pltpu

_E = 8
_K = 2
_D = 1024
_F = 2816
_R = 16
_T = 4096
_ER = _E * _R
_TILE = 256
_NT = _T // _TILE


def _mix_tile(x, wr, wg, wu, ag, au, bg, bu, adt):
    f32 = jnp.float32
    m = x.shape[0]

    logits = jnp.dot(x, wr, preferred_element_type=f32)
    eidx = jax.lax.broadcasted_iota(jnp.int32, (m, _E), 1)
    l0 = jnp.max(logits, axis=-1, keepdims=True)
    i0 = jnp.min(jnp.where(logits == l0, eidx, _E), axis=-1, keepdims=True)
    masked = jnp.where(eidx == i0, -jnp.inf, logits)
    l1 = jnp.max(masked, axis=-1, keepdims=True)
    i1 = jnp.min(jnp.where(masked == l1, eidx, _E), axis=-1, keepdims=True)
    w0 = jax.nn.sigmoid(l0 - l1)
    w1 = 1.0 - w0

    g0 = jnp.dot(x, wg, preferred_element_type=f32)
    u0 = jnp.dot(x, wu, preferred_element_type=f32)
    dn_rt = (((1,), (1,)), ((), ()))
    pg = jax.lax.dot_general(x, ag, dn_rt, preferred_element_type=f32)
    pu = jax.lax.dot_general(x, au, dn_rt, preferred_element_type=f32)

    cidx = jax.lax.broadcasted_iota(jnp.int32, (m, _ER), 1) // _R
    hbar = jnp.zeros((m, _F), f32)
    q = jnp.zeros((m, _ER), f32)
    for ik, wk in ((i0, w0), (i1, w1)):
        mk = cidx == ik
        g = g0 + jnp.dot(jnp.where(mk, pg, 0.0), bg, preferred_element_type=f32)
        u = u0 + jnp.dot(jnp.where(mk, pu, 0.0), bu, preferred_element_type=f32)
        wh = (g * jax.nn.sigmoid(g)) * u * wk
        hbar = hbar + wh
        qf = jax.lax.dot_general(wh, adt, (((1,), (1,)), ((), ())),
                                 preferred_element_type=f32)
        q = q + jnp.where(mk, qf, 0.0)
    return hbar, q


def _fused(x_ref, wr_ref, wg_ref, wu_ref, ag_ref, bg_ref, au_ref, bu_ref,
           adt_ref, wd_ref, bd_ref, out_ref, hb_last_ref, qq_last_ref,
           hb_ref, qq_ref):
    i = pl.program_id(0)
    cur = jax.lax.rem(i, 2)
    prev = 1 - cur

    f32 = jnp.float32
    x = x_ref[...]
    wr = wr_ref[...]
    bg = bg_ref[...]
    bu = bu_ref[...]
    adt = adt_ref[...]
    m = _TILE

    logits = jnp.dot(x, wr, preferred_element_type=f32)
    eidx = jax.lax.broadcasted_iota(jnp.int32, (m, _E), 1)
    l0 = jnp.max(logits, axis=-1, keepdims=True)
    i0 = jnp.min(jnp.where(logits == l0, eidx, _E), axis=-1, keepdims=True)
    masked = jnp.where(eidx == i0, -jnp.inf, logits)
    l1 = jnp.max(masked, axis=-1, keepdims=True)
    i1 = jnp.min(jnp.where(masked == l1, eidx, _E), axis=-1, keepdims=True)
    w0 = jax.nn.sigmoid(l0 - l1)
    w1 = 1.0 - w0

    g0 = jnp.dot(x, wg_ref[...], preferred_element_type=f32)
    u0 = jnp.dot(x, wu_ref[...], preferred_element_type=f32)
    dn_rt = (((1,), (1,)), ((), ()))
    pg = jax.lax.dot_general(x, ag_ref[...], dn_rt, preferred_element_type=f32)
    pu = jax.lax.dot_general(x, au_ref[...], dn_rt, preferred_element_type=f32)

    out_ref[...] = (
        jnp.dot(hb_ref[prev], wd_ref[...], preferred_element_type=jnp.float32)
        + jnp.dot(qq_ref[prev], bd_ref[...], preferred_element_type=jnp.float32))

    cidx = jax.lax.broadcasted_iota(jnp.int32, (m, _ER), 1) // _R
    hbar = jnp.zeros((m, _F), f32)
    q = jnp.zeros((m, _ER), f32)
    for ik, wk in ((i0, w0), (i1, w1)):
        mk = cidx == ik
        g = g0 + jnp.dot(jnp.where(mk, pg, 0.0), bg, preferred_element_type=f32)
        u = u0 + jnp.dot(jnp.where(mk, pu, 0.0), bu, preferred_element_type=f32)
        wh = (g * jax.nn.sigmoid(g)) * u * wk
        hbar = hbar + wh
        qf = jax.lax.dot_general(wh, adt, (((1,), (1,)), ((), ())),
                                 preferred_element_type=f32)
        q = q + jnp.where(mk, qf, 0.0)
    hbar16 = hbar.astype(jnp.bfloat16)
    q16 = q.astype(jnp.bfloat16)
    hb_ref[cur] = hbar16
    qq_ref[cur] = q16
    hb_last_ref[...] = hbar16
    qq_last_ref[...] = q16


def _last_tile_down(hb_ref, qq_ref, wd_ref, bd_ref, outin_ref, out_ref):
    out_ref[...] = (
        jnp.dot(hb_ref[...], wd_ref[...], preferred_element_type=jnp.float32)
        + jnp.dot(qq_ref[...], bd_ref[...], preferred_element_type=jnp.float32))


def kernel(hidden_states, W_router, W_gate, W_up, W_down,
           A_gate, B_gate, A_up, B_up, A_down, B_down):
    f32 = jnp.float32
    ag = A_gate.reshape(_ER, _D)
    au = A_up.reshape(_ER, _D)
    bg = B_gate.transpose(0, 2, 1).reshape(_ER, _F)
    bu = B_up.transpose(0, 2, 1).reshape(_ER, _F)
    adt = A_down.reshape(_ER, _F)
    bd = B_down.transpose(0, 2, 1).reshape(_ER, _D)

    const = lambda i: (0, 0)
    out, hb_last, qq_last = pl.pallas_call(
        _fused,
        grid=(_NT,),
        in_specs=[
            pl.BlockSpec((_TILE, _D), lambda i: (i, 0)),
            pl.BlockSpec((_D, _E), const),
            pl.BlockSpec((_D, _F), const),
            pl.BlockSpec((_D, _F), const),
            pl.BlockSpec((_ER, _D), const),
            pl.BlockSpec((_ER, _F), const),
            pl.BlockSpec((_ER, _D), const),
            pl.BlockSpec((_ER, _F), const),
            pl.BlockSpec((_ER, _F), const),
            pl.BlockSpec((_F, _D), const),
            pl.BlockSpec((_ER, _D), const),
        ],
        out_specs=[
            pl.BlockSpec((_TILE, _D), lambda i: (jnp.maximum(i - 1, 0), 0)),
            pl.BlockSpec((_TILE, _F), lambda i: (0, 0)),
            pl.BlockSpec((_TILE, _ER), lambda i: (0, 0)),
        ],
        out_shape=[
            jax.ShapeDtypeStruct((_T, _D), f32),
            jax.ShapeDtypeStruct((_TILE, _F), jnp.bfloat16),
            jax.ShapeDtypeStruct((_TILE, _ER), jnp.bfloat16),
        ],
        scratch_shapes=[
            pltpu.VMEM((2, _TILE, _F), jnp.bfloat16),
            pltpu.VMEM((2, _TILE, _ER), jnp.bfloat16),
        ],
    )(hidden_states, W_router, W_gate, W_up, ag, bg, au, bu, adt, W_down, bd)

    return pl.pallas_call(
        _last_tile_down,
        grid=(1,),
        in_specs=[
            pl.BlockSpec((_TILE, _F), const),
            pl.BlockSpec((_TILE, _ER), const),
            pl.BlockSpec((_F, _D), const),
            pl.BlockSpec((_ER, _D), const),
            pl.BlockSpec((_TILE, _D), lambda i: (_NT - 1, 0)),
        ],
        out_specs=pl.BlockSpec((_TILE, _D), lambda i: (_NT - 1, 0)),
        out_shape=jax.ShapeDtypeStruct((_T, _D), f32),
        input_output_aliases={4: 0},
    )(hb_last, qq_last, W_down, bd, out)
